# Initial kernel scaffold; baseline (speedup 1.0000x reference)
import jax
import jax.numpy as jnp
from jax.experimental import pallas as pl


def kernel(batch_ids, table):
    raise NotImplementedError("write your pallas kernel here")


# speedup vs baseline: 1.1031x; 1.1031x over previous
import functools

import jax
import jax.numpy as jnp
from jax import lax
from jax.experimental import pallas as pl
from jax.experimental.pallas import tpu as pltpu
from jax.experimental.pallas import tpu_sc as plsc


def _gather_call(B, D, b_per_w, chunk, nchunks, NC):
    mesh = plsc.VectorSubcoreMesh(core_axis_name="c", subcore_axis_name="s")

    @functools.partial(
        pl.kernel,
        mesh=mesh,
        out_type=jax.ShapeDtypeStruct((B, D), jnp.float32),
        scratch_types=[
            pltpu.VMEM((chunk,), jnp.int32),
            pltpu.VMEM((chunk, D), jnp.float32),
            pltpu.SemaphoreType.DMA,
        ],
        compiler_params=pltpu.CompilerParams(use_tc_tiling_on_sc=False),
    )
    def k(idx_hbm, table_hbm, out_hbm, idx_v, rows_v, sem):
        wid = lax.axis_index("s") * NC + lax.axis_index("c")
        base = wid * b_per_w

        def body(c, carry):
            off = base + c * chunk
            pltpu.sync_copy(idx_hbm.at[pl.ds(off, chunk)], idx_v)
            pltpu.async_copy(table_hbm.at[idx_v], rows_v, sem).wait()
            pltpu.sync_copy(rows_v, out_hbm.at[pl.ds(off, chunk)])
            return carry

        lax.fori_loop(0, nchunks, body, 0)

    return k


def kernel(batch_ids, table):
    B0, H = batch_ids.shape
    V, D = table.shape
    B = B0 * H
    flat = batch_ids.reshape(B).astype(jnp.int32)

    info = plsc.get_sparse_core_info()
    NC, NS = info.num_cores, info.num_subcores
    NW = NC * NS
    b_per_w = B // NW
    chunk = 1600
    nchunks = b_per_w // chunk

    out = _gather_call(B, D, b_per_w, chunk, nchunks, NC)(flat, table)
    return out.reshape(B0, H, D)


# speedup vs baseline: 1.1117x; 1.0079x over previous
import functools

import jax
import jax.numpy as jnp
from jax import lax
from jax.experimental import pallas as pl
from jax.experimental.pallas import tpu as pltpu
from jax.experimental.pallas import tpu_sc as plsc


def _gather_call(B, D, b_per_w, chunk, nchunks, NC):
    mesh = plsc.VectorSubcoreMesh(core_axis_name="c", subcore_axis_name="s")

    @functools.partial(
        pl.kernel,
        mesh=mesh,
        out_type=jax.ShapeDtypeStruct((B, D), jnp.float32),
        scratch_types=[
            pltpu.VMEM((chunk,), jnp.int32),
            pltpu.VMEM((chunk,), jnp.int32),
            pltpu.VMEM((chunk, D), jnp.float32),
            pltpu.VMEM((chunk, D), jnp.float32),
            pltpu.SemaphoreType.DMA,
            pltpu.SemaphoreType.DMA,
            pltpu.SemaphoreType.DMA,
            pltpu.SemaphoreType.DMA,
        ],
        compiler_params=pltpu.CompilerParams(use_tc_tiling_on_sc=False),
    )
    def k(idx_hbm, table_hbm, out_hbm, i0, i1, r0, r1, sg0, sg1, so0, so1):
        idx_v = [i0, i1]
        rows_v = [r0, r1]
        sg = [sg0, sg1]
        so = [so0, so1]
        wid = lax.axis_index("s") * NC + lax.axis_index("c")
        base = wid * b_per_w

        gathers = [None, None]
        outs = [None, None]
        pltpu.sync_copy(idx_hbm.at[pl.ds(base, chunk)], idx_v[0])
        gathers[0] = pltpu.async_copy(table_hbm.at[idx_v[0]], rows_v[0], sg[0])
        for c in range(nchunks):
            b = c % 2
            nb = (c + 1) % 2
            if c + 1 < nchunks:
                off = base + (c + 1) * chunk
                pltpu.sync_copy(idx_hbm.at[pl.ds(off, chunk)], idx_v[nb])
                if c >= 1:
                    outs[nb].wait()
                gathers[nb] = pltpu.async_copy(
                    table_hbm.at[idx_v[nb]], rows_v[nb], sg[nb])
            gathers[b].wait()
            outs[b] = pltpu.async_copy(
                rows_v[b], out_hbm.at[pl.ds(base + c * chunk, chunk)], so[b])
        outs[0].wait()
        outs[1].wait()

    return k


def kernel(batch_ids, table):
    B0, H = batch_ids.shape
    V, D = table.shape
    B = B0 * H
    flat = batch_ids.reshape(B).astype(jnp.int32)

    info = plsc.get_sparse_core_info()
    NC, NS = info.num_cores, info.num_subcores
    NW = NC * NS
    b_per_w = B // NW
    chunk = 1600
    nchunks = b_per_w // chunk

    out = _gather_call(B, D, b_per_w, chunk, nchunks, NC)(flat, table)
    return out.reshape(B0, H, D)


# speedup vs baseline: 1.8111x; 1.6291x over previous
import functools

import jax
import jax.numpy as jnp
from jax import lax
from jax.experimental import pallas as pl
from jax.experimental.pallas import tpu as pltpu
from jax.experimental.pallas import tpu_sc as plsc


def _gather_call(B0, H, D, bs_per_w, bchunk, nchunks, NC):
    mesh = plsc.VectorSubcoreMesh(core_axis_name="c", subcore_axis_name="s")
    chunk = bchunk * H

    @functools.partial(
        pl.kernel,
        mesh=mesh,
        out_type=jax.ShapeDtypeStruct((B0, H, D), jnp.float32),
        scratch_types=[
            pltpu.VMEM((chunk,), jnp.int32),
            pltpu.VMEM((chunk,), jnp.int32),
            pltpu.VMEM((chunk, D), jnp.float32),
            pltpu.VMEM((chunk, D), jnp.float32),
            pltpu.SemaphoreType.DMA,
            pltpu.SemaphoreType.DMA,
            pltpu.SemaphoreType.DMA,
            pltpu.SemaphoreType.DMA,
        ],
        compiler_params=pltpu.CompilerParams(use_tc_tiling_on_sc=False),
    )
    def k(idx_hbm, table_hbm, out_hbm, i0, i1, r0, r1, sg0, sg1, so0, so1):
        idx_v = [i0, i1]
        rows_v = [r0, r1]
        sg = [sg0, sg1]
        so = [so0, so1]
        wid = lax.axis_index("s") * NC + lax.axis_index("c")
        b_base = wid * bs_per_w

        def issue_outs(b, c):
            b0 = b_base + c * bchunk

            def body(i, carry):
                pltpu.async_copy(
                    rows_v[b].at[pl.ds(i * H, H)], out_hbm.at[b0 + i], so[b])
                return carry

            lax.fori_loop(0, bchunk, body, 0)
            return pltpu.make_async_copy(
                table_hbm.at[pl.ds(0, chunk)], rows_v[b], so[b])

        gathers = [None, None]
        outs = [None, None]
        pltpu.sync_copy(idx_hbm.at[pl.ds(b_base * H, chunk)], idx_v[0])
        gathers[0] = pltpu.async_copy(table_hbm.at[idx_v[0]], rows_v[0], sg[0])
        for c in range(nchunks):
            b = c % 2
            nb = (c + 1) % 2
            if c + 1 < nchunks:
                off = (b_base + (c + 1) * bchunk) * H
                pltpu.sync_copy(idx_hbm.at[pl.ds(off, chunk)], idx_v[nb])
                if c >= 1:
                    outs[nb].wait()
                gathers[nb] = pltpu.async_copy(
                    table_hbm.at[idx_v[nb]], rows_v[nb], sg[nb])
            gathers[b].wait()
            outs[b] = issue_outs(b, c)
        outs[0].wait()
        outs[1].wait()

    return k


def kernel(batch_ids, table):
    B0, H = batch_ids.shape
    V, D = table.shape
    flat = batch_ids.reshape(B0 * H).astype(jnp.int32)

    info = plsc.get_sparse_core_info()
    NC, NS = info.num_cores, info.num_subcores
    NW = NC * NS
    bs_per_w = B0 // NW
    bchunk = 32
    nchunks = bs_per_w // bchunk

    return _gather_call(B0, H, D, bs_per_w, bchunk, nchunks, NC)(flat, table)
